# baseline (speedup 1.0000x reference)
import functools
import math

import jax
import jax.numpy as jnp
from jax import lax
from jax.experimental import pallas as pl
from jax.experimental.pallas import tpu as pltpu
from jax.experimental.pallas import tpu_sc as plsc

N = 10000
E = 160000
B = 64
DIM = 256
NFEAT = 19
NG = 50
CUTOFF = 5.0

NPAD = 10240
CH = 80
NCHUNKS = E // CH
ROWS_PER_SUB = NPAD // 16

_mesh = functools.partial(
    plsc.VectorSubcoreMesh, core_axis_name="c", subcore_axis_name="s",
    num_cores=2, num_subcores=16)


def _dotT(a, w):
    return lax.dot_general(a, w, (((1,), (1,)), ((), ())),
                           preferred_element_type=jnp.float32)


def _sc_degree(dst, ones_rows, zeros_rows):
    @functools.partial(
        pl.kernel,
        out_type=jax.ShapeDtypeStruct((NPAD, DIM), jnp.float32),
        mesh=_mesh(),
        scratch_types=[
            pltpu.VMEM((CH,), jnp.int32),
            pltpu.VMEM((CH, 128), jnp.float32),
            pltpu.VMEM_SHARED((NPAD, 128), jnp.float32),
        ],
    )
    def k(dst_hbm, ones_hbm, zeros_hbm, out_hbm, idx_v, ones_v, acc):
        c = lax.axis_index("c")
        s = lax.axis_index("s")
        w = s * 2 + c
        rowbase = s * ROWS_PER_SUB
        pltpu.sync_copy(zeros_hbm, acc.at[pl.ds(rowbase, ROWS_PER_SUB)])
        pltpu.sync_copy(ones_hbm, ones_v)
        plsc.subcore_barrier()
        nc32, rem32 = NCHUNKS // 32, NCHUNKS % 32
        nch = nc32 + jnp.where(w < rem32, 1, 0)

        def body(j, carry):
            cid = w + j * 32
            pltpu.sync_copy(dst_hbm.at[pl.ds(cid * CH, CH)], idx_v)
            pltpu.sync_copy(ones_v, acc.at[idx_v], add=True)
            return carry

        lax.fori_loop(0, nch, body, 0)
        plsc.subcore_barrier()
        pltpu.sync_copy(acc.at[pl.ds(rowbase, ROWS_PER_SUB)],
                        out_hbm.at[pl.ds(rowbase, ROWS_PER_SUB), pl.ds(c * 128, 128)])

    return k(dst, ones_rows, zeros_rows)


def _sc_cfconv(xi, wfilt, src, dst, zeros_rows):
    npersub = NCHUNKS // 16
    G = 5
    ngroups = npersub // G

    @functools.partial(
        pl.kernel,
        out_type=jax.ShapeDtypeStruct((NPAD, DIM), jnp.float32),
        mesh=_mesh(),
        scratch_types=[
            pltpu.VMEM((2, G, CH), jnp.int32),
            pltpu.VMEM((2, G, CH), jnp.int32),
            pltpu.VMEM((2, CH, 128), jnp.float32),
            pltpu.VMEM((2, CH, 128), jnp.float32),
            pltpu.VMEM_SHARED((NPAD, 128), jnp.float32),
            pltpu.SemaphoreType.DMA,
            pltpu.SemaphoreType.DMA,
            pltpu.SemaphoreType.DMA,
            pltpu.SemaphoreType.DMA,
        ],
    )
    def k(xi_hbm, wf_hbm, src_hbm, dst_hbm, zeros_hbm, out_hbm,
          sidx_v, didx_v, xrows_v, wrows_v, acc, dsem0, dsem1, isem0, isem1):
        c = lax.axis_index("c")
        s = lax.axis_index("s")
        colbase = c * 128
        rowbase = s * ROWS_PER_SUB
        dsems = (dsem0, dsem1)
        isems = (isem0, isem1)
        cbase = s * npersub
        pltpu.sync_copy(zeros_hbm, acc.at[pl.ds(rowbase, ROWS_PER_SUB)])
        plsc.subcore_barrier()

        def fetch_idx(g, slot):
            gid = s * ngroups + g
            pltpu.async_copy(src_hbm.at[gid], sidx_v.at[slot], isems[slot])
            pltpu.async_copy(dst_hbm.at[gid], didx_v.at[slot], isems[slot])

        def drain_idx(slot):
            pltpu.make_async_copy(src_hbm.at[0], sidx_v.at[slot],
                                  isems[slot]).wait()
            pltpu.make_async_copy(src_hbm.at[0], didx_v.at[slot],
                                  isems[slot]).wait()

        def issue_data(g, islot, k_in_g, dslot):
            cid = cbase + g * G + k_in_g
            pltpu.async_copy(
                xi_hbm.at[sidx_v.at[islot, k_in_g], pl.ds(colbase, 128)],
                xrows_v.at[dslot], dsems[dslot])
            pltpu.async_copy(
                wf_hbm.at[pl.ds(cid * CH, CH), pl.ds(colbase, 128)],
                wrows_v.at[dslot], dsems[dslot])

        def process(islot, k_in_g, dslot):
            pltpu.make_async_copy(
                wf_hbm.at[pl.ds(0, CH), pl.ds(0, 128)], xrows_v.at[dslot],
                dsems[dslot]).wait()
            pltpu.make_async_copy(
                wf_hbm.at[pl.ds(0, CH), pl.ds(0, 128)], wrows_v.at[dslot],
                dsems[dslot]).wait()

            def mul_row(r, carry2):
                for rr in range(2):
                    for l in range(8):
                        sl = pl.ds(l * 16, 16)
                        wrows_v[dslot, 2 * r + rr, sl] = (
                            wrows_v[dslot, 2 * r + rr, sl]
                            * xrows_v[dslot, 2 * r + rr, sl])
                return carry2

            lax.fori_loop(0, CH // 2, mul_row, 0)
            pltpu.sync_copy(wrows_v.at[dslot],
                            acc.at[didx_v.at[islot, k_in_g]], add=True)

        fetch_idx(0, 0)
        drain_idx(0)
        fetch_idx(1, 1)
        issue_data(0, 0, 0, 0)

        def body(g, carry):
            gslot_flags = (lax.rem(g, 2) == 0, lax.rem(g, 2) == 1)
            for islot in range(2):
                @pl.when(gslot_flags[islot])
                def _(islot=islot):
                    for kk in range(G - 1):
                        issue_data(g, islot, kk + 1, (islot + kk + 1) % 2)
                        process(islot, kk, (islot + kk) % 2)

                    @pl.when(g + 1 < ngroups)
                    def _():
                        drain_idx(1 - islot)
                        issue_data(g + 1, 1 - islot, 0, 1 - islot)
                    process(islot, G - 1, (islot + G - 1) % 2)
                    @pl.when(g + 2 < ngroups)
                    def _():
                        fetch_idx(g + 2, islot)
            return carry

        lax.fori_loop(0, ngroups, body, 0)

        plsc.subcore_barrier()
        pltpu.sync_copy(acc.at[pl.ds(rowbase, ROWS_PER_SUB)],
                        out_hbm.at[pl.ds(rowbase, ROWS_PER_SUB), pl.ds(colbase, 128)])

    return k(xi, wfilt, src, dst, zeros_rows)


_NODE_BLK = 2000
_EDGE_BLK = 2000


def _full(shape):
    return pl.BlockSpec(shape, lambda i: tuple(0 for _ in shape))


def _filt_kernel(ew_ref, wn1_ref, bn1_ref, wn2_ref, bn2_ref, o_ref):
    ew = ew_ref[...]
    step = CUTOFF / (NG - 1)
    offset = lax.broadcasted_iota(jnp.int32, (1, NG), 1).astype(jnp.float32) * step
    coeff = -0.5 / (step * step)
    d = ew - offset
    ea = jnp.exp(coeff * d * d).astype(jnp.bfloat16)
    h1 = jnp.maximum(
        _dotT(ea, wn1_ref[...].astype(jnp.bfloat16)) + bn1_ref[...], 0.0)
    cfac = (jnp.cos(ew * (math.pi / CUTOFF)) + 1.0) * 0.5
    o_ref[...] = (_dotT(h1.astype(jnp.bfloat16),
                        wn2_ref[...].astype(jnp.bfloat16))
                  + bn2_ref[...]) * cfac


def _tc_filter(ew2, Wn1, bn1, Wn2, bn2):
    grid = E // _EDGE_BLK
    return pl.pallas_call(
        _filt_kernel,
        grid=(grid,),
        in_specs=[
            pl.BlockSpec((_EDGE_BLK, 1), lambda i: (i, 0)),
            _full((128, NG)), _full((1, 128)),
            _full((DIM, 128)), _full((1, DIM)),
        ],
        out_specs=pl.BlockSpec((_EDGE_BLK, DIM), lambda i: (i, 0)),
        out_shape=jax.ShapeDtypeStruct((E, DIM), jnp.float32),
    )(ew2, Wn1, bn1, Wn2, bn2)


def _lin0_kernel(x_ref, w_ref, b_ref, wc1_ref, h_ref, xi_ref):
    h = jnp.maximum(_dotT(x_ref[...], w_ref[...]) + b_ref[...], 0.0)
    h_ref[...] = h
    xi_ref[...] = _dotT(h, wc1_ref[...])


def _tc_lin0(x, W_lin0, b_lin0, Wc1):
    grid = N // _NODE_BLK
    return pl.pallas_call(
        _lin0_kernel,
        grid=(grid,),
        in_specs=[
            pl.BlockSpec((_NODE_BLK, NFEAT), lambda i: (i, 0)),
            _full((DIM, NFEAT)), _full((1, DIM)), _full((DIM, DIM)),
        ],
        out_specs=[
            pl.BlockSpec((_NODE_BLK, DIM), lambda i: (i, 0)),
            pl.BlockSpec((_NODE_BLK, DIM), lambda i: (i, 0)),
        ],
        out_shape=[
            jax.ShapeDtypeStruct((N, DIM), jnp.float32),
            jax.ShapeDtypeStruct((N, DIM), jnp.float32),
        ],
    )(x, W_lin0, b_lin0, Wc1)


def _big_kernel(agg_ref, deg_ref, h_ref, wc2_ref, bc2_ref, wc3_ref,
                wih_ref, bih_ref, whh_ref, bhh_ref, wc1_ref,
                h_out_ref, xi_out_ref):
    log2 = math.log(2.0)
    degm = deg_ref[...]
    d = jnp.maximum(degm[:, 0:1] + degm[:, 128:129], 1.0)
    agg = agg_ref[...] / d
    t = _dotT(agg, wc2_ref[...]) + bc2_ref[...]
    t = jnp.maximum(t, 0.0) + jnp.log(1.0 + jnp.exp(-jnp.abs(t))) - log2
    m = jnp.maximum(_dotT(t, wc3_ref[...]), 0.0)
    gi = _dotT(m, wih_ref[...]) + bih_ref[...]
    h = h_ref[...]
    gh = _dotT(h, whh_ref[...]) + bhh_ref[...]
    r = jax.nn.sigmoid(gi[:, 0:DIM] + gh[:, 0:DIM])
    z = jax.nn.sigmoid(gi[:, DIM:2 * DIM] + gh[:, DIM:2 * DIM])
    n = jnp.tanh(gi[:, 2 * DIM:] + r * gh[:, 2 * DIM:])
    h_new = (1.0 - z) * n + z * h
    h_out_ref[...] = h_new
    xi_out_ref[...] = _dotT(h_new, wc1_ref[...])


def _tc_big(aggp, degp, h, Wc2, bc2, Wc3, W_ih, b_ih, W_hh, b_hh, Wc1):
    grid = N // _NODE_BLK
    nspec = pl.BlockSpec((_NODE_BLK, DIM), lambda i: (i, 0))
    return pl.pallas_call(
        _big_kernel,
        grid=(grid,),
        in_specs=[
            nspec,
            nspec,
            nspec,
            _full((DIM, DIM)), _full((1, DIM)), _full((DIM, DIM)),
            _full((3 * DIM, DIM)), _full((1, 3 * DIM)),
            _full((3 * DIM, DIM)), _full((1, 3 * DIM)),
            _full((DIM, DIM)),
        ],
        out_specs=[nspec, nspec],
        out_shape=[
            jax.ShapeDtypeStruct((N, DIM), jnp.float32),
            jax.ShapeDtypeStruct((N, DIM), jnp.float32),
        ],
    )(aggp, degp, h, Wc2, bc2, Wc3, W_ih, b_ih, W_hh, b_hh, Wc1)


def _s2s_kernel(h_ref, b_ref, wsih_ref, bsih_ref, wshh_ref, bshh_ref,
                wl1_ref, bl1_ref, wl2_ref, o_ref):
    hmat = h_ref[...]
    bcol = b_ref[...]
    onehot = (bcol == lax.broadcasted_iota(jnp.int32, (1, B), 1))
    qstar = jnp.zeros((B, 2 * DIM), jnp.float32)
    hs = jnp.zeros((B, DIM), jnp.float32)
    cs = jnp.zeros((B, DIM), jnp.float32)
    for _ in range(3):
        gates = (_dotT(qstar, wsih_ref[...]) + bsih_ref[...]
                 + _dotT(hs, wshh_ref[...]) + bshh_ref[...])
        i_g = jax.nn.sigmoid(gates[:, 0:DIM])
        f_g = jax.nn.sigmoid(gates[:, DIM:2 * DIM])
        g_g = jnp.tanh(gates[:, 2 * DIM:3 * DIM])
        o_g = jax.nn.sigmoid(gates[:, 3 * DIM:])
        cs = f_g * cs + i_g * g_g
        hs = o_g * jnp.tanh(cs)
        q = hs
        em = _dotT(hmat, q)
        emax = jnp.max(jnp.where(onehot, em, -1e30), axis=0, keepdims=True)
        a = jnp.where(onehot, jnp.exp(em - emax), 0.0)
        asum = jnp.sum(a, axis=0, keepdims=True)
        an = a / (asum + 1e-16)
        r = lax.dot_general(an, hmat, (((0,), (0,)), ((), ())),
                            preferred_element_type=jnp.float32)
        qstar = jnp.concatenate([q, r], axis=1)
    o = jnp.maximum(_dotT(qstar, wl1_ref[...]) + bl1_ref[...], 0.0)
    o_ref[...] = jnp.sum(o * wl2_ref[...], axis=1, keepdims=True)


def _tc_set2set(h, batch2, Ws_ih, bs_ih, Ws_hh, bs_hh, Wl1, bl1, Wl2):
    return pl.pallas_call(
        _s2s_kernel,
        out_shape=jax.ShapeDtypeStruct((B, 1), jnp.float32),
    )(h, batch2, Ws_ih, bs_ih, Ws_hh, bs_hh, Wl1, bl1, Wl2)


def kernel(x, edge_index, edge_weight, batch, W_lin0, b_lin0, Wn1, bn1, Wn2,
           bn2, Wc1, Wc2, bc2, Wc3, W_ih, W_hh, b_ih, b_hh, Ws_ih, Ws_hh,
           bs_ih, bs_hh, Wl1, bl1, Wl2, bl2):
    src = edge_index[0]
    dst = edge_index[1]
    ew2 = edge_weight.reshape(E, 1)
    batch2 = batch.reshape(N, 1)
    ones128 = jnp.ones((CH, 128), jnp.float32)
    zeros128 = jnp.zeros((ROWS_PER_SUB, 128), jnp.float32)

    wfilt = _tc_filter(ew2, Wn1, bn1.reshape(1, -1), Wn2, bn2.reshape(1, -1))
    h, xi = _tc_lin0(x, W_lin0, b_lin0.reshape(1, -1), Wc1)
    degp = _sc_degree(dst, ones128, zeros128)

    src3 = src.reshape(NCHUNKS // 5, 5, CH)
    dst3 = dst.reshape(NCHUNKS // 5, 5, CH)
    for _ in range(3):
        aggp = _sc_cfconv(xi, wfilt, src3, dst3, zeros128)
        h, xi = _tc_big(aggp, degp, h, Wc2, bc2.reshape(1, -1), Wc3,
                        W_ih, b_ih.reshape(1, -1), W_hh, b_hh.reshape(1, -1),
                        Wc1)

    o = _tc_set2set(h, batch2, Ws_ih, bs_ih.reshape(1, -1), Ws_hh,
                    bs_hh.reshape(1, -1), Wl1, bl1.reshape(1, -1), Wl2)
    return o.reshape(-1) + bl2


# speedup vs baseline: 1.0145x; 1.0145x over previous
import functools
import math

import jax
import jax.numpy as jnp
from jax import lax
from jax.experimental import pallas as pl
from jax.experimental.pallas import tpu as pltpu
from jax.experimental.pallas import tpu_sc as plsc

N = 10000
E = 160000
B = 64
DIM = 256
NFEAT = 19
NG = 50
CUTOFF = 5.0

NPAD = 10240
CH = 80
NCHUNKS = E // CH
ROWS_PER_SUB = NPAD // 16

_mesh = functools.partial(
    plsc.VectorSubcoreMesh, core_axis_name="c", subcore_axis_name="s",
    num_cores=2, num_subcores=16)


def _dotT(a, w):
    return lax.dot_general(a, w, (((1,), (1,)), ((), ())),
                           preferred_element_type=jnp.float32)


def _sc_degree(dst, ones_rows, zeros_rows):
    @functools.partial(
        pl.kernel,
        out_type=jax.ShapeDtypeStruct((NPAD, DIM), jnp.float32),
        mesh=_mesh(),
        scratch_types=[
            pltpu.VMEM((CH,), jnp.int32),
            pltpu.VMEM((CH, 128), jnp.float32),
            pltpu.VMEM_SHARED((NPAD, 128), jnp.float32),
        ],
    )
    def k(dst_hbm, ones_hbm, zeros_hbm, out_hbm, idx_v, ones_v, acc):
        c = lax.axis_index("c")
        s = lax.axis_index("s")
        w = s * 2 + c
        rowbase = s * ROWS_PER_SUB
        pltpu.sync_copy(zeros_hbm, acc.at[pl.ds(rowbase, ROWS_PER_SUB)])
        pltpu.sync_copy(ones_hbm, ones_v)
        plsc.subcore_barrier()
        nc32, rem32 = NCHUNKS // 32, NCHUNKS % 32
        nch = nc32 + jnp.where(w < rem32, 1, 0)

        def body(j, carry):
            cid = w + j * 32
            pltpu.sync_copy(dst_hbm.at[pl.ds(cid * CH, CH)], idx_v)
            pltpu.sync_copy(ones_v, acc.at[idx_v], add=True)
            return carry

        lax.fori_loop(0, nch, body, 0)
        plsc.subcore_barrier()
        pltpu.sync_copy(acc.at[pl.ds(rowbase, ROWS_PER_SUB)],
                        out_hbm.at[pl.ds(rowbase, ROWS_PER_SUB), pl.ds(c * 128, 128)])

    return k(dst, ones_rows, zeros_rows)


def _sc_cfconv(xi, wfilt, src, dst, zeros_rows):
    npersub = NCHUNKS // 16
    G = 5
    ngroups = npersub // G

    @functools.partial(
        pl.kernel,
        out_type=jax.ShapeDtypeStruct((NPAD, DIM), jnp.float32),
        mesh=_mesh(),
        scratch_types=[
            pltpu.VMEM((2, G, CH), jnp.int32),
            pltpu.VMEM((2, G, CH), jnp.int32),
            pltpu.VMEM((2, CH, 128), jnp.float32),
            pltpu.VMEM((2, CH, 128), jnp.float32),
            pltpu.VMEM_SHARED((NPAD, 128), jnp.float32),
            pltpu.SemaphoreType.DMA,
            pltpu.SemaphoreType.DMA,
            pltpu.SemaphoreType.DMA,
            pltpu.SemaphoreType.DMA,
        ],
    )
    def k(xi_hbm, wf_hbm, src_hbm, dst_hbm, zeros_hbm, out_hbm,
          sidx_v, didx_v, xrows_v, wrows_v, acc, dsem0, dsem1, isem0, isem1):
        c = lax.axis_index("c")
        s = lax.axis_index("s")
        colbase = c * 128
        rowbase = s * ROWS_PER_SUB
        dsems = (dsem0, dsem1)
        isems = (isem0, isem1)
        cbase = s * npersub
        pltpu.sync_copy(zeros_hbm, acc.at[pl.ds(rowbase, ROWS_PER_SUB)])
        plsc.subcore_barrier()

        def fetch_idx(g, slot):
            gid = s * ngroups + g
            pltpu.async_copy(src_hbm.at[gid], sidx_v.at[slot], isems[slot])
            pltpu.async_copy(dst_hbm.at[gid], didx_v.at[slot], isems[slot])

        def drain_idx(slot):
            pltpu.make_async_copy(src_hbm.at[0], sidx_v.at[slot],
                                  isems[slot]).wait()
            pltpu.make_async_copy(src_hbm.at[0], didx_v.at[slot],
                                  isems[slot]).wait()

        def issue_data(g, islot, k_in_g, dslot):
            cid = cbase + g * G + k_in_g
            pltpu.async_copy(
                xi_hbm.at[sidx_v.at[islot, k_in_g], pl.ds(colbase, 128)],
                xrows_v.at[dslot], dsems[dslot])
            pltpu.async_copy(
                wf_hbm.at[pl.ds(cid * CH, CH), pl.ds(colbase, 128)],
                wrows_v.at[dslot], dsems[dslot])

        def process(islot, k_in_g, dslot):
            pltpu.make_async_copy(
                wf_hbm.at[pl.ds(0, CH), pl.ds(0, 128)], xrows_v.at[dslot],
                dsems[dslot]).wait()
            pltpu.make_async_copy(
                wf_hbm.at[pl.ds(0, CH), pl.ds(0, 128)], wrows_v.at[dslot],
                dsems[dslot]).wait()

            def mul_row(r, carry2):
                for rr in range(2):
                    for l in range(8):
                        sl = pl.ds(l * 16, 16)
                        wrows_v[dslot, 2 * r + rr, sl] = (
                            wrows_v[dslot, 2 * r + rr, sl]
                            * xrows_v[dslot, 2 * r + rr, sl])
                return carry2

            lax.fori_loop(0, CH // 2, mul_row, 0)
            pltpu.sync_copy(wrows_v.at[dslot],
                            acc.at[didx_v.at[islot, k_in_g]], add=True)

        fetch_idx(0, 0)
        drain_idx(0)
        fetch_idx(1, 1)
        issue_data(0, 0, 0, 0)

        def body(g, carry):
            gslot_flags = (lax.rem(g, 2) == 0, lax.rem(g, 2) == 1)
            for islot in range(2):
                @pl.when(gslot_flags[islot])
                def _(islot=islot):
                    for kk in range(G - 1):
                        issue_data(g, islot, kk + 1, (islot + kk + 1) % 2)
                        process(islot, kk, (islot + kk) % 2)

                    @pl.when(g + 1 < ngroups)
                    def _():
                        drain_idx(1 - islot)
                        issue_data(g + 1, 1 - islot, 0, 1 - islot)
                    process(islot, G - 1, (islot + G - 1) % 2)
                    @pl.when(g + 2 < ngroups)
                    def _():
                        fetch_idx(g + 2, islot)
            return carry

        lax.fori_loop(0, ngroups, body, 0)

        plsc.subcore_barrier()
        pltpu.sync_copy(acc.at[pl.ds(rowbase, ROWS_PER_SUB)],
                        out_hbm.at[pl.ds(rowbase, ROWS_PER_SUB), pl.ds(colbase, 128)])

    return k(xi, wfilt, src, dst, zeros_rows)


_NODE_BLK = 2000
_EDGE_BLK = 2000


def _full(shape):
    return pl.BlockSpec(shape, lambda i: tuple(0 for _ in shape))


def _filt_kernel(ew_ref, wn1_ref, bn1_ref, wn2_ref, bn2_ref, o_ref):
    ew = ew_ref[...]
    step = CUTOFF / (NG - 1)
    offset = lax.broadcasted_iota(jnp.int32, (1, NG), 1).astype(jnp.float32) * step
    coeff = -0.5 / (step * step)
    d = ew - offset
    ea = jnp.exp(coeff * d * d)
    h1 = jnp.maximum(_dotT(ea, wn1_ref[...]) + bn1_ref[...], 0.0)
    cfac = (jnp.cos(ew * (math.pi / CUTOFF)) + 1.0) * 0.5
    o_ref[...] = (_dotT(h1, wn2_ref[...]) + bn2_ref[...]) * cfac


def _tc_filter(ew2, Wn1, bn1, Wn2, bn2):
    grid = E // _EDGE_BLK
    return pl.pallas_call(
        _filt_kernel,
        grid=(grid,),
        in_specs=[
            pl.BlockSpec((_EDGE_BLK, 1), lambda i: (i, 0)),
            _full((128, NG)), _full((1, 128)),
            _full((DIM, 128)), _full((1, DIM)),
        ],
        out_specs=pl.BlockSpec((_EDGE_BLK, DIM), lambda i: (i, 0)),
        out_shape=jax.ShapeDtypeStruct((E, DIM), jnp.float32),
    )(ew2, Wn1, bn1, Wn2, bn2)


def _lin0_kernel(x_ref, w_ref, b_ref, wc1_ref, h_ref, xi_ref):
    h = jnp.maximum(_dotT(x_ref[...], w_ref[...]) + b_ref[...], 0.0)
    h_ref[...] = h
    xi_ref[...] = _dotT(h, wc1_ref[...])


def _tc_lin0(x, W_lin0, b_lin0, Wc1):
    grid = N // _NODE_BLK
    return pl.pallas_call(
        _lin0_kernel,
        grid=(grid,),
        in_specs=[
            pl.BlockSpec((_NODE_BLK, NFEAT), lambda i: (i, 0)),
            _full((DIM, NFEAT)), _full((1, DIM)), _full((DIM, DIM)),
        ],
        out_specs=[
            pl.BlockSpec((_NODE_BLK, DIM), lambda i: (i, 0)),
            pl.BlockSpec((_NODE_BLK, DIM), lambda i: (i, 0)),
        ],
        out_shape=[
            jax.ShapeDtypeStruct((N, DIM), jnp.float32),
            jax.ShapeDtypeStruct((N, DIM), jnp.float32),
        ],
    )(x, W_lin0, b_lin0, Wc1)


def _big_kernel(agg_ref, deg_ref, h_ref, wc2_ref, bc2_ref, wc3_ref,
                wih_ref, bih_ref, whh_ref, bhh_ref, wc1_ref,
                h_out_ref, xi_out_ref):
    log2 = math.log(2.0)
    degm = deg_ref[...]
    d = jnp.maximum(degm[:, 0:1] + degm[:, 128:129], 1.0)
    agg = agg_ref[...] / d
    t = _dotT(agg, wc2_ref[...]) + bc2_ref[...]
    t = jnp.maximum(t, 0.0) + jnp.log(1.0 + jnp.exp(-jnp.abs(t))) - log2
    m = jnp.maximum(_dotT(t, wc3_ref[...]), 0.0)
    gi = _dotT(m, wih_ref[...]) + bih_ref[...]
    h = h_ref[...]
    gh = _dotT(h, whh_ref[...]) + bhh_ref[...]
    r = jax.nn.sigmoid(gi[:, 0:DIM] + gh[:, 0:DIM])
    z = jax.nn.sigmoid(gi[:, DIM:2 * DIM] + gh[:, DIM:2 * DIM])
    n = jnp.tanh(gi[:, 2 * DIM:] + r * gh[:, 2 * DIM:])
    h_new = (1.0 - z) * n + z * h
    h_out_ref[...] = h_new
    xi_out_ref[...] = _dotT(h_new, wc1_ref[...])


def _tc_big(aggp, degp, h, Wc2, bc2, Wc3, W_ih, b_ih, W_hh, b_hh, Wc1):
    grid = N // _NODE_BLK
    nspec = pl.BlockSpec((_NODE_BLK, DIM), lambda i: (i, 0))
    return pl.pallas_call(
        _big_kernel,
        grid=(grid,),
        in_specs=[
            nspec,
            nspec,
            nspec,
            _full((DIM, DIM)), _full((1, DIM)), _full((DIM, DIM)),
            _full((3 * DIM, DIM)), _full((1, 3 * DIM)),
            _full((3 * DIM, DIM)), _full((1, 3 * DIM)),
            _full((DIM, DIM)),
        ],
        out_specs=[nspec, nspec],
        out_shape=[
            jax.ShapeDtypeStruct((N, DIM), jnp.float32),
            jax.ShapeDtypeStruct((N, DIM), jnp.float32),
        ],
    )(aggp, degp, h, Wc2, bc2, Wc3, W_ih, b_ih, W_hh, b_hh, Wc1)


def _s2s_kernel(h_ref, b_ref, wsih_ref, bsih_ref, wshh_ref, bshh_ref,
                wl1_ref, bl1_ref, wl2_ref, o_ref):
    hmat = h_ref[...]
    bcol = b_ref[...]
    onehot = (bcol == lax.broadcasted_iota(jnp.int32, (1, B), 1))
    qstar = jnp.zeros((B, 2 * DIM), jnp.float32)
    hs = jnp.zeros((B, DIM), jnp.float32)
    cs = jnp.zeros((B, DIM), jnp.float32)
    for _ in range(3):
        gates = (_dotT(qstar, wsih_ref[...]) + bsih_ref[...]
                 + _dotT(hs, wshh_ref[...]) + bshh_ref[...])
        i_g = jax.nn.sigmoid(gates[:, 0:DIM])
        f_g = jax.nn.sigmoid(gates[:, DIM:2 * DIM])
        g_g = jnp.tanh(gates[:, 2 * DIM:3 * DIM])
        o_g = jax.nn.sigmoid(gates[:, 3 * DIM:])
        cs = f_g * cs + i_g * g_g
        hs = o_g * jnp.tanh(cs)
        q = hs
        em = _dotT(hmat, q)
        emax = jnp.max(jnp.where(onehot, em, -1e30), axis=0, keepdims=True)
        a = jnp.where(onehot, jnp.exp(em - emax), 0.0)
        asum = jnp.sum(a, axis=0, keepdims=True)
        an = a / (asum + 1e-16)
        r = lax.dot_general(an, hmat, (((0,), (0,)), ((), ())),
                            preferred_element_type=jnp.float32)
        qstar = jnp.concatenate([q, r], axis=1)
    o = jnp.maximum(_dotT(qstar, wl1_ref[...]) + bl1_ref[...], 0.0)
    o_ref[...] = jnp.sum(o * wl2_ref[...], axis=1, keepdims=True)


def _tc_set2set(h, batch2, Ws_ih, bs_ih, Ws_hh, bs_hh, Wl1, bl1, Wl2):
    return pl.pallas_call(
        _s2s_kernel,
        out_shape=jax.ShapeDtypeStruct((B, 1), jnp.float32),
    )(h, batch2, Ws_ih, bs_ih, Ws_hh, bs_hh, Wl1, bl1, Wl2)


def kernel(x, edge_index, edge_weight, batch, W_lin0, b_lin0, Wn1, bn1, Wn2,
           bn2, Wc1, Wc2, bc2, Wc3, W_ih, W_hh, b_ih, b_hh, Ws_ih, Ws_hh,
           bs_ih, bs_hh, Wl1, bl1, Wl2, bl2):
    src = edge_index[0]
    dst = edge_index[1]
    ew2 = edge_weight.reshape(E, 1)
    batch2 = batch.reshape(N, 1)
    ones128 = jnp.ones((CH, 128), jnp.float32)
    zeros128 = jnp.zeros((ROWS_PER_SUB, 128), jnp.float32)

    degp = _sc_degree(dst, ones128, zeros128)
    wfilt = _tc_filter(ew2, Wn1, bn1.reshape(1, -1), Wn2, bn2.reshape(1, -1))
    h, xi = _tc_lin0(x, W_lin0, b_lin0.reshape(1, -1), Wc1)

    src3 = src.reshape(NCHUNKS // 5, 5, CH)
    dst3 = dst.reshape(NCHUNKS // 5, 5, CH)
    for _ in range(3):
        aggp = _sc_cfconv(xi, wfilt, src3, dst3, zeros128)
        h, xi = _tc_big(aggp, degp, h, Wc2, bc2.reshape(1, -1), Wc3,
                        W_ih, b_ih.reshape(1, -1), W_hh, b_hh.reshape(1, -1),
                        Wc1)

    o = _tc_set2set(h, batch2, Ws_ih, bs_ih.reshape(1, -1), Ws_hh,
                    bs_hh.reshape(1, -1), Wl1, bl1.reshape(1, -1), Wl2)
    return o.reshape(-1) + bl2


# speedup vs baseline: 1.1476x; 1.1312x over previous
import functools
import math

import jax
import jax.numpy as jnp
from jax import lax
from jax.experimental import pallas as pl
from jax.experimental.pallas import tpu as pltpu
from jax.experimental.pallas import tpu_sc as plsc

N = 10000
E = 160000
B = 64
DIM = 256
NFEAT = 19
NG = 50
CUTOFF = 5.0

NPAD = 10240
CH = 80
NCHUNKS = E // CH
ROWS_PER_SUB = NPAD // 16

_mesh = functools.partial(
    plsc.VectorSubcoreMesh, core_axis_name="c", subcore_axis_name="s",
    num_cores=2, num_subcores=16)


def _dotT(a, w):
    return lax.dot_general(a, w, (((1,), (1,)), ((), ())),
                           preferred_element_type=jnp.float32)


def _sc_degree(dst, ones_rows, zeros_rows):
    @functools.partial(
        pl.kernel,
        out_type=jax.ShapeDtypeStruct((NPAD, DIM), jnp.float32),
        mesh=_mesh(),
        scratch_types=[
            pltpu.VMEM((CH,), jnp.int32),
            pltpu.VMEM((CH, 128), jnp.float32),
            pltpu.VMEM_SHARED((NPAD, 128), jnp.float32),
        ],
    )
    def k(dst_hbm, ones_hbm, zeros_hbm, out_hbm, idx_v, ones_v, acc):
        c = lax.axis_index("c")
        s = lax.axis_index("s")
        w = s * 2 + c
        rowbase = s * ROWS_PER_SUB
        pltpu.sync_copy(zeros_hbm, acc.at[pl.ds(rowbase, ROWS_PER_SUB)])
        pltpu.sync_copy(ones_hbm, ones_v)
        plsc.subcore_barrier()
        nc32, rem32 = NCHUNKS // 32, NCHUNKS % 32
        nch = nc32 + jnp.where(w < rem32, 1, 0)

        def body(j, carry):
            cid = w + j * 32
            pltpu.sync_copy(dst_hbm.at[pl.ds(cid * CH, CH)], idx_v)
            pltpu.sync_copy(ones_v, acc.at[idx_v], add=True)
            return carry

        lax.fori_loop(0, nch, body, 0)
        plsc.subcore_barrier()
        pltpu.sync_copy(acc.at[pl.ds(rowbase, ROWS_PER_SUB)],
                        out_hbm.at[pl.ds(rowbase, ROWS_PER_SUB), pl.ds(c * 128, 128)])

    return k(dst, ones_rows, zeros_rows)


def _sc_cfconv(xi, wfilt, src, dst, zeros_rows):
    npersub = NCHUNKS // 16
    G = 5
    ngroups = npersub // G

    @functools.partial(
        pl.kernel,
        out_type=jax.ShapeDtypeStruct((NPAD, DIM), jnp.float32),
        mesh=_mesh(),
        scratch_types=[
            pltpu.VMEM((2, G, CH), jnp.int32),
            pltpu.VMEM((2, G, CH), jnp.int32),
            pltpu.VMEM((2, CH, 128), jnp.float32),
            pltpu.VMEM((2, CH, 128), jnp.float32),
            pltpu.VMEM_SHARED((NPAD, 128), jnp.float32),
            pltpu.SemaphoreType.DMA,
            pltpu.SemaphoreType.DMA,
            pltpu.SemaphoreType.DMA,
            pltpu.SemaphoreType.DMA,
        ],
    )
    def k(xi_hbm, wf_hbm, src_hbm, dst_hbm, zeros_hbm, out_hbm,
          sidx_v, didx_v, xrows_v, wrows_v, acc, dsem0, dsem1, isem0, isem1):
        c = lax.axis_index("c")
        s = lax.axis_index("s")
        colbase = c * 128
        rowbase = s * ROWS_PER_SUB
        dsems = (dsem0, dsem1)
        isems = (isem0, isem1)
        cbase = s * npersub
        pltpu.sync_copy(zeros_hbm, acc.at[pl.ds(rowbase, ROWS_PER_SUB)])
        plsc.subcore_barrier()

        def fetch_idx(g, slot):
            gid = s * ngroups + g
            pltpu.async_copy(src_hbm.at[gid], sidx_v.at[slot], isems[slot])
            pltpu.async_copy(dst_hbm.at[gid], didx_v.at[slot], isems[slot])

        def drain_idx(slot):
            pltpu.make_async_copy(src_hbm.at[0], sidx_v.at[slot],
                                  isems[slot]).wait()
            pltpu.make_async_copy(src_hbm.at[0], didx_v.at[slot],
                                  isems[slot]).wait()

        def issue_data(g, islot, k_in_g, dslot):
            cid = cbase + g * G + k_in_g
            pltpu.async_copy(
                xi_hbm.at[sidx_v.at[islot, k_in_g], pl.ds(colbase, 128)],
                xrows_v.at[dslot], dsems[dslot])
            pltpu.async_copy(
                wf_hbm.at[pl.ds(cid * CH, CH), pl.ds(colbase, 128)],
                wrows_v.at[dslot], dsems[dslot])

        def process(islot, k_in_g, dslot):
            pltpu.make_async_copy(
                wf_hbm.at[pl.ds(0, CH), pl.ds(0, 128)], xrows_v.at[dslot],
                dsems[dslot]).wait()
            pltpu.make_async_copy(
                wf_hbm.at[pl.ds(0, CH), pl.ds(0, 128)], wrows_v.at[dslot],
                dsems[dslot]).wait()

            def mul_row(r, carry2):
                for rr in range(2):
                    for l in range(8):
                        sl = pl.ds(l * 16, 16)
                        wrows_v[dslot, 2 * r + rr, sl] = (
                            wrows_v[dslot, 2 * r + rr, sl]
                            * xrows_v[dslot, 2 * r + rr, sl])
                return carry2

            lax.fori_loop(0, CH // 2, mul_row, 0)
            pltpu.sync_copy(wrows_v.at[dslot],
                            acc.at[didx_v.at[islot, k_in_g]], add=True)

        fetch_idx(0, 0)
        drain_idx(0)
        fetch_idx(1, 1)
        issue_data(0, 0, 0, 0)

        def body(g, carry):
            gslot_flags = (lax.rem(g, 2) == 0, lax.rem(g, 2) == 1)
            for islot in range(2):
                @pl.when(gslot_flags[islot])
                def _(islot=islot):
                    for kk in range(G - 1):
                        issue_data(g, islot, kk + 1, (islot + kk + 1) % 2)
                        process(islot, kk, (islot + kk) % 2)

                    @pl.when(g + 1 < ngroups)
                    def _():
                        drain_idx(1 - islot)
                        issue_data(g + 1, 1 - islot, 0, 1 - islot)
                    process(islot, G - 1, (islot + G - 1) % 2)
                    @pl.when(g + 2 < ngroups)
                    def _():
                        fetch_idx(g + 2, islot)
            return carry

        lax.fori_loop(0, ngroups, body, 0)

        plsc.subcore_barrier()
        pltpu.sync_copy(acc.at[pl.ds(rowbase, ROWS_PER_SUB)],
                        out_hbm.at[pl.ds(rowbase, ROWS_PER_SUB), pl.ds(colbase, 128)])

    return k(xi, wfilt, src, dst, zeros_rows)


_NODE_BLK = 2000
_EDGE_BLK = 2000


def _full(shape):
    return pl.BlockSpec(shape, lambda i: tuple(0 for _ in shape))


def _cfac_kernel(ew_ref, o_ref):
    o_ref[...] = (jnp.cos(ew_ref[...] * (math.pi / CUTOFF)) + 1.0) * 0.5


def _tc_cfac(ew_wide):
    return pl.pallas_call(
        _cfac_kernel,
        out_shape=jax.ShapeDtypeStruct(ew_wide.shape, jnp.float32),
    )(ew_wide)


def _filt_kernel(ew_ref, cf_ref, wn1_ref, bn1_ref, wn2_ref, bn2_ref, o_ref):
    ew = ew_ref[...]
    step = CUTOFF / (NG - 1)
    offset = lax.broadcasted_iota(jnp.int32, (1, NG), 1).astype(jnp.float32) * step
    coeff = -0.5 / (step * step)
    d = ew - offset
    ea = jnp.exp(coeff * d * d)
    h1 = jnp.maximum(_dotT(ea, wn1_ref[...]) + bn1_ref[...], 0.0)
    o_ref[...] = (_dotT(h1, wn2_ref[...]) + bn2_ref[...]) * cf_ref[...]


def _tc_filter(ew2, cf2, Wn1, bn1, Wn2, bn2):
    grid = E // _EDGE_BLK
    return pl.pallas_call(
        _filt_kernel,
        grid=(grid,),
        in_specs=[
            pl.BlockSpec((_EDGE_BLK, 1), lambda i: (i, 0)),
            pl.BlockSpec((_EDGE_BLK, 1), lambda i: (i, 0)),
            _full((128, NG)), _full((1, 128)),
            _full((DIM, 128)), _full((1, DIM)),
        ],
        out_specs=pl.BlockSpec((_EDGE_BLK, DIM), lambda i: (i, 0)),
        out_shape=jax.ShapeDtypeStruct((E, DIM), jnp.float32),
    )(ew2, cf2, Wn1, bn1, Wn2, bn2)


def _lin0_kernel(x_ref, w_ref, b_ref, wc1_ref, h_ref, xi_ref):
    h = jnp.maximum(_dotT(x_ref[...], w_ref[...]) + b_ref[...], 0.0)
    h_ref[...] = h
    xi_ref[...] = _dotT(h, wc1_ref[...])


def _tc_lin0(x, W_lin0, b_lin0, Wc1):
    grid = N // _NODE_BLK
    return pl.pallas_call(
        _lin0_kernel,
        grid=(grid,),
        in_specs=[
            pl.BlockSpec((_NODE_BLK, NFEAT), lambda i: (i, 0)),
            _full((DIM, NFEAT)), _full((1, DIM)), _full((DIM, DIM)),
        ],
        out_specs=[
            pl.BlockSpec((_NODE_BLK, DIM), lambda i: (i, 0)),
            pl.BlockSpec((_NODE_BLK, DIM), lambda i: (i, 0)),
        ],
        out_shape=[
            jax.ShapeDtypeStruct((N, DIM), jnp.float32),
            jax.ShapeDtypeStruct((N, DIM), jnp.float32),
        ],
    )(x, W_lin0, b_lin0, Wc1)


def _big_kernel(agg_ref, deg_ref, h_ref, wc2_ref, bc2_ref, wc3_ref,
                wih_ref, bih_ref, whh_ref, bhh_ref, wc1_ref,
                h_out_ref, xi_out_ref):
    log2 = math.log(2.0)
    degm = deg_ref[...]
    d = jnp.maximum(degm[:, 0:1] + degm[:, 128:129], 1.0)
    agg = agg_ref[...] / d
    t = _dotT(agg, wc2_ref[...]) + bc2_ref[...]
    t = jnp.maximum(t, 0.0) + jnp.log(1.0 + jnp.exp(-jnp.abs(t))) - log2
    m = jnp.maximum(_dotT(t, wc3_ref[...]), 0.0)
    gi = _dotT(m, wih_ref[...]) + bih_ref[...]
    h = h_ref[...]
    gh = _dotT(h, whh_ref[...]) + bhh_ref[...]
    r = jax.nn.sigmoid(gi[:, 0:DIM] + gh[:, 0:DIM])
    z = jax.nn.sigmoid(gi[:, DIM:2 * DIM] + gh[:, DIM:2 * DIM])
    n = jnp.tanh(gi[:, 2 * DIM:] + r * gh[:, 2 * DIM:])
    h_new = (1.0 - z) * n + z * h
    h_out_ref[...] = h_new
    xi_out_ref[...] = _dotT(h_new, wc1_ref[...])


def _tc_big(aggp, degp, h, Wc2, bc2, Wc3, W_ih, b_ih, W_hh, b_hh, Wc1):
    grid = N // _NODE_BLK
    nspec = pl.BlockSpec((_NODE_BLK, DIM), lambda i: (i, 0))
    return pl.pallas_call(
        _big_kernel,
        grid=(grid,),
        in_specs=[
            nspec,
            nspec,
            nspec,
            _full((DIM, DIM)), _full((1, DIM)), _full((DIM, DIM)),
            _full((3 * DIM, DIM)), _full((1, 3 * DIM)),
            _full((3 * DIM, DIM)), _full((1, 3 * DIM)),
            _full((DIM, DIM)),
        ],
        out_specs=[nspec, nspec],
        out_shape=[
            jax.ShapeDtypeStruct((N, DIM), jnp.float32),
            jax.ShapeDtypeStruct((N, DIM), jnp.float32),
        ],
    )(aggp, degp, h, Wc2, bc2, Wc3, W_ih, b_ih, W_hh, b_hh, Wc1)


def _s2s_kernel(h_ref, b_ref, wsih_ref, bsih_ref, wshh_ref, bshh_ref,
                wl1_ref, bl1_ref, wl2_ref, o_ref):
    hmat = h_ref[...]
    bcol = b_ref[...]
    onehot = (bcol == lax.broadcasted_iota(jnp.int32, (1, B), 1))
    qstar = jnp.zeros((B, 2 * DIM), jnp.float32)
    hs = jnp.zeros((B, DIM), jnp.float32)
    cs = jnp.zeros((B, DIM), jnp.float32)
    for _ in range(3):
        gates = (_dotT(qstar, wsih_ref[...]) + bsih_ref[...]
                 + _dotT(hs, wshh_ref[...]) + bshh_ref[...])
        i_g = jax.nn.sigmoid(gates[:, 0:DIM])
        f_g = jax.nn.sigmoid(gates[:, DIM:2 * DIM])
        g_g = jnp.tanh(gates[:, 2 * DIM:3 * DIM])
        o_g = jax.nn.sigmoid(gates[:, 3 * DIM:])
        cs = f_g * cs + i_g * g_g
        hs = o_g * jnp.tanh(cs)
        q = hs
        em = _dotT(hmat, q)
        emax = jnp.max(jnp.where(onehot, em, -1e30), axis=0, keepdims=True)
        a = jnp.where(onehot, jnp.exp(em - emax), 0.0)
        asum = jnp.sum(a, axis=0, keepdims=True)
        an = a / (asum + 1e-16)
        r = lax.dot_general(an, hmat, (((0,), (0,)), ((), ())),
                            preferred_element_type=jnp.float32)
        qstar = jnp.concatenate([q, r], axis=1)
    o = jnp.maximum(_dotT(qstar, wl1_ref[...]) + bl1_ref[...], 0.0)
    o_ref[...] = jnp.sum(o * wl2_ref[...], axis=1, keepdims=True)


def _tc_set2set(h, batch2, Ws_ih, bs_ih, Ws_hh, bs_hh, Wl1, bl1, Wl2):
    return pl.pallas_call(
        _s2s_kernel,
        out_shape=jax.ShapeDtypeStruct((B, 1), jnp.float32),
    )(h, batch2, Ws_ih, bs_ih, Ws_hh, bs_hh, Wl1, bl1, Wl2)


def kernel(x, edge_index, edge_weight, batch, W_lin0, b_lin0, Wn1, bn1, Wn2,
           bn2, Wc1, Wc2, bc2, Wc3, W_ih, W_hh, b_ih, b_hh, Ws_ih, Ws_hh,
           bs_ih, bs_hh, Wl1, bl1, Wl2, bl2):
    src = edge_index[0]
    dst = edge_index[1]
    ew2 = edge_weight.reshape(E, 1)
    batch2 = batch.reshape(N, 1)
    ones128 = jnp.ones((CH, 128), jnp.float32)
    zeros128 = jnp.zeros((ROWS_PER_SUB, 128), jnp.float32)

    degp = _sc_degree(dst, ones128, zeros128)
    cf2 = _tc_cfac(edge_weight.reshape(E // 128, 128)).reshape(E, 1)
    wfilt = _tc_filter(ew2, cf2, Wn1, bn1.reshape(1, -1), Wn2, bn2.reshape(1, -1))
    h, xi = _tc_lin0(x, W_lin0, b_lin0.reshape(1, -1), Wc1)

    src3 = src.reshape(NCHUNKS // 5, 5, CH)
    dst3 = dst.reshape(NCHUNKS // 5, 5, CH)
    for _ in range(3):
        aggp = _sc_cfconv(xi, wfilt, src3, dst3, zeros128)
        h, xi = _tc_big(aggp, degp, h, Wc2, bc2.reshape(1, -1), Wc3,
                        W_ih, b_ih.reshape(1, -1), W_hh, b_hh.reshape(1, -1),
                        Wc1)

    o = _tc_set2set(h, batch2, Ws_ih, bs_ih.reshape(1, -1), Ws_hh,
                    bs_hh.reshape(1, -1), Wl1, bl1.reshape(1, -1), Wl2)
    return o.reshape(-1) + bl2
